# baseline (speedup 1.0000x reference)
import functools

import jax
import jax.numpy as jnp
from jax import lax
from jax.experimental import pallas as pl
from jax.experimental.pallas import tpu as pltpu
from jax.experimental.pallas import tpu_sc as plsc

LATENT = 64
XDIM = 16
UDIM = LATENT + XDIM
LANES = 16
IDX_CHUNK = 128


def kernel(users, items, xij, emb_user, emb_item, emb_item_xij1, emb_item_xij0):
    B = users.shape[0]
    info = plsc.get_sparse_core_info()
    NC, NS = info.num_cores, info.num_subcores
    NW = NC * NS
    assert B % (NW * LANES) == 0
    RPW = B // NW
    NCHUNK = RPW // IDX_CHUNK

    mesh = plsc.VectorSubcoreMesh(core_axis_name="c", subcore_axis_name="s")

    @functools.partial(
        pl.kernel,
        out_type=jax.ShapeDtypeStruct((B,), jnp.float32),
        mesh=mesh,
        scratch_types=[
            pltpu.VMEM((NCHUNK, IDX_CHUNK), jnp.int32),
            pltpu.VMEM((NCHUNK, IDX_CHUNK), jnp.int32),
            pltpu.VMEM((RPW,), jnp.float32),
            pltpu.VMEM((RPW, UDIM), jnp.float32),
            pltpu.VMEM((RPW, LATENT), jnp.float32),
            pltpu.VMEM((RPW, XDIM), jnp.float32),
            pltpu.VMEM((RPW, XDIM), jnp.float32),
            pltpu.VMEM((RPW,), jnp.float32),
            pltpu.SemaphoreType.DMA,
        ],
        compiler_params=pltpu.CompilerParams(
            needs_layout_passes=False, use_tc_tiling_on_sc=False),
    )
    def sc_kernel(users_h, items_h, xij_h, eu_h, ei_h, e1_h, e0_h, out_h,
                  uidx, iidx, xv, urows, irows, x1rows, x0rows, outv, sem):
        wid = lax.axis_index("s") * NC + lax.axis_index("c")
        base = wid * RPW

        for j in range(NCHUNK):
            sl = pl.ds(base + j * IDX_CHUNK, IDX_CHUNK)
            pltpu.sync_copy(users_h.at[sl], uidx.at[j])
            pltpu.sync_copy(items_h.at[sl], iidx.at[j])
        pltpu.sync_copy(xij_h.at[pl.ds(base, RPW)], xv)

        copies = []
        for j in range(NCHUNK):
            sl = pl.ds(j * IDX_CHUNK, IDX_CHUNK)
            copies.append(pltpu.async_copy(eu_h.at[uidx.at[j]], urows.at[sl], sem))
            copies.append(pltpu.async_copy(ei_h.at[iidx.at[j]], irows.at[sl], sem))
            copies.append(pltpu.async_copy(e1_h.at[iidx.at[j]], x1rows.at[sl], sem))
            copies.append(pltpu.async_copy(e0_h.at[iidx.at[j]], x0rows.at[sl], sem))
        for c in copies:
            c.wait()

        lanes = lax.broadcasted_iota(jnp.int32, (LANES,), 0)
        one = jnp.float32(1.0)
        def group_body(g, carry):
            rows = g * LANES + lanes
            x = xv[pl.ds(g * LANES, LANES)]
            denom = jnp.zeros((LANES,), jnp.float32)
            numer = jnp.zeros((LANES,), jnp.float32)
            for d in range(LATENT):
                dd = jnp.full((LANES,), d, jnp.int32)
                z = plsc.load_gather(irows, [rows, dd])
                e = jnp.exp(z)
                u = plsc.load_gather(urows, [rows, dd])
                s = one / (one + jnp.exp(-u))
                denom = denom + e
                numer = numer + s * e
            for d in range(XDIM):
                dd = jnp.full((LANES,), d, jnp.int32)
                x1 = plsc.load_gather(x1rows, [rows, dd])
                x0 = plsc.load_gather(x0rows, [rows, dd])
                z = x1 * x + x0 * (one - x)
                e = jnp.exp(z)
                du = jnp.full((LANES,), LATENT + d, jnp.int32)
                u = plsc.load_gather(urows, [rows, du])
                s = one / (one + jnp.exp(-u))
                denom = denom + e
                numer = numer + s * e
            outv[pl.ds(g * LANES, LANES)] = numer / denom
            return carry

        lax.fori_loop(0, RPW // LANES, group_body, 0)
        pltpu.sync_copy(outv, out_h.at[pl.ds(base, RPW)])

    return sc_kernel(users.astype(jnp.int32), items, xij, emb_user,
                     emb_item, emb_item_xij1, emb_item_xij0)


# speedup vs baseline: 1.1297x; 1.1297x over previous
import functools

import jax
import jax.numpy as jnp
from jax import lax
from jax.experimental import pallas as pl
from jax.experimental.pallas import tpu as pltpu
from jax.experimental.pallas import tpu_sc as plsc

LATENT = 64
XDIM = 16
UDIM = LATENT + XDIM
LANES = 16
WIDTH = 128
IDX_CHUNK = 128
HALF = 256


def kernel(users, items, xij, emb_user, emb_item, emb_item_xij1, emb_item_xij0):
    B = users.shape[0]
    NI = emb_item.shape[0]
    info = plsc.get_sparse_core_info()
    NC, NS = info.num_cores, info.num_subcores
    NW = NC * NS
    assert B % (NW * LANES) == 0
    RPW = B // NW
    NCHUNK = RPW // IDX_CHUNK
    NHALF = RPW // HALF
    CPH = HALF // IDX_CHUNK

    eu_p = jnp.pad(emb_user, ((0, 0), (0, WIDTH - UDIM)))
    icat = jnp.concatenate(
        [emb_item, emb_item_xij1, emb_item_xij0,
         jnp.zeros((NI, WIDTH - LATENT - 2 * XDIM), jnp.float32)], axis=1)

    mesh = plsc.VectorSubcoreMesh(core_axis_name="c", subcore_axis_name="s")

    @functools.partial(
        pl.kernel,
        out_type=jax.ShapeDtypeStruct((B,), jnp.float32),
        mesh=mesh,
        scratch_types=[
            pltpu.VMEM((NCHUNK, IDX_CHUNK), jnp.int32),
            pltpu.VMEM((NCHUNK, IDX_CHUNK), jnp.int32),
            pltpu.VMEM((RPW,), jnp.float32),
            pltpu.VMEM((HALF, WIDTH), jnp.float32),
            pltpu.VMEM((HALF, WIDTH), jnp.float32),
            pltpu.VMEM((RPW,), jnp.float32),
            pltpu.SemaphoreType.DMA,
        ],
        compiler_params=pltpu.CompilerParams(
            needs_layout_passes=False, use_tc_tiling_on_sc=True),
    )
    def sc_kernel(users_h, items_h, xij_h, eu_h, ic_h, out_h,
                  uidx, iidx, xv, ubuf, ibuf, outv, sem):
        wid = lax.axis_index("s") * NC + lax.axis_index("c")
        base = wid * RPW

        for j in range(NCHUNK):
            sl = pl.ds(base + j * IDX_CHUNK, IDX_CHUNK)
            pltpu.sync_copy(users_h.at[sl], uidx.at[j])
            pltpu.sync_copy(items_h.at[sl], iidx.at[j])
        pltpu.sync_copy(xij_h.at[pl.ds(base, RPW)], xv)

        lanes = lax.broadcasted_iota(jnp.int32, (LANES,), 0)
        one = jnp.float32(1.0)

        for h in range(NHALF):
            copies = []
            for c in range(CPH):
                j = h * CPH + c
                sl = pl.ds(c * IDX_CHUNK, IDX_CHUNK)
                copies.append(pltpu.async_copy(eu_h.at[uidx.at[j]],
                                               ubuf.at[sl], sem))
                copies.append(pltpu.async_copy(ic_h.at[iidx.at[j]],
                                               ibuf.at[sl], sem))
            for cp in copies:
                cp.wait()

            def group_body(g, carry):
                rows = g * LANES + lanes
                x = xv[pl.ds(h * HALF + g * LANES, LANES)]
                denom = jnp.zeros((LANES,), jnp.float32)
                numer = jnp.zeros((LANES,), jnp.float32)
                for d in range(LATENT):
                    dd = jnp.full((LANES,), d, jnp.int32)
                    z = plsc.load_gather(ibuf, [rows, dd])
                    e = jnp.exp(z)
                    u = plsc.load_gather(ubuf, [rows, dd])
                    s = one / (one + jnp.exp(-u))
                    denom = denom + e
                    numer = numer + s * e
                for d in range(XDIM):
                    d1 = jnp.full((LANES,), LATENT + d, jnp.int32)
                    d0 = jnp.full((LANES,), LATENT + XDIM + d, jnp.int32)
                    x1 = plsc.load_gather(ibuf, [rows, d1])
                    x0 = plsc.load_gather(ibuf, [rows, d0])
                    z = x1 * x + x0 * (one - x)
                    e = jnp.exp(z)
                    u = plsc.load_gather(ubuf, [rows, d1])
                    s = one / (one + jnp.exp(-u))
                    denom = denom + e
                    numer = numer + s * e
                outv[pl.ds(h * HALF + g * LANES, LANES)] = numer / denom
                return carry

            lax.fori_loop(0, HALF // LANES, group_body, 0)

        pltpu.sync_copy(outv, out_h.at[pl.ds(base, RPW)])

    return sc_kernel(users.astype(jnp.int32), items, xij, eu_p, icat)
